# baseline (speedup 1.0000x reference)
import functools

import jax
import jax.numpy as jnp
from jax.experimental import pallas as pl
from jax.experimental.pallas import tpu as pltpu


def _hop1_body(x16_ref, adj_ref, adj16_ref, y1_ref, y116_ref):
    a16 = adj_ref[...].astype(jnp.bfloat16)
    adj16_ref[...] = a16
    y1 = jnp.dot(a16, x16_ref[...], preferred_element_type=jnp.float32)
    y1_ref[...] = y1
    y116_ref[...] = y1.astype(jnp.bfloat16)


def _hop2_body(adj16_ref, x_ref, y1_ref, y116_ref, w_ref, bg_ref, pt_ref,
               bp_ref, h_ref, z_ref):
    y2 = jnp.dot(adj16_ref[...], y116_ref[...],
                 preferred_element_type=jnp.float32)
    xm = (x_ref[...] + y1_ref[...] + y2) * (1.0 / 3.0)
    hb = jnp.dot(xm, w_ref[...], preferred_element_type=jnp.float32)
    hb = jnp.maximum(hb + bg_ref[...], 0.0)
    h_ref[...] = hb
    zl = jnp.dot(hb, pt_ref[...], preferred_element_type=jnp.float32)
    zl = zl + bp_ref[...]
    m = jnp.max(zl, axis=1, keepdims=True)
    e = zl - m
    lse = jnp.log(jnp.sum(jnp.exp(e), axis=1, keepdims=True))
    z_ref[...] = e - lse


def kernel(x, adj, W, b_gcn, prototypes, b_pre, t_p):
    n, din = x.shape
    dh = W.shape[1]
    dout = prototypes.shape[0]

    bm1 = 200
    while n % bm1:
        bm1 //= 2
    bm2 = 400
    while n % bm2:
        bm2 //= 2

    inv_t = (1.0 / t_p).astype(jnp.float32) if hasattr(t_p, "astype") else jnp.float32(1.0 / t_p)
    pt = prototypes.T.astype(jnp.float32) * inv_t
    bp = (b_pre.astype(jnp.float32) * inv_t).reshape(1, dout)
    bg = b_gcn.reshape(1, dh)
    x16 = x.astype(jnp.bfloat16)

    adj16, y1, y116 = pl.pallas_call(
        _hop1_body,
        grid=(n // bm1,),
        in_specs=[
            pl.BlockSpec((n, din), lambda i: (0, 0)),
            pl.BlockSpec((bm1, n), lambda i: (i, 0)),
        ],
        out_specs=[
            pl.BlockSpec((bm1, n), lambda i: (i, 0)),
            pl.BlockSpec((bm1, din), lambda i: (i, 0)),
            pl.BlockSpec((bm1, din), lambda i: (i, 0)),
        ],
        out_shape=[
            jax.ShapeDtypeStruct((n, n), jnp.bfloat16),
            jax.ShapeDtypeStruct((n, din), jnp.float32),
            jax.ShapeDtypeStruct((n, din), jnp.bfloat16),
        ],
        compiler_params=pltpu.CompilerParams(
            dimension_semantics=("arbitrary",),
            vmem_limit_bytes=56 * 1024 * 1024,
        ),
    )(x16, adj)

    h, z = pl.pallas_call(
        _hop2_body,
        grid=(n // bm2,),
        in_specs=[
            pl.BlockSpec((bm2, n), lambda i: (i, 0)),
            pl.BlockSpec((bm2, din), lambda i: (i, 0)),
            pl.BlockSpec((bm2, din), lambda i: (i, 0)),
            pl.BlockSpec((n, din), lambda i: (0, 0)),
            pl.BlockSpec((din, dh), lambda i: (0, 0)),
            pl.BlockSpec((1, dh), lambda i: (0, 0)),
            pl.BlockSpec((dh, dout), lambda i: (0, 0)),
            pl.BlockSpec((1, dout), lambda i: (0, 0)),
        ],
        out_specs=[
            pl.BlockSpec((bm2, dh), lambda i: (i, 0)),
            pl.BlockSpec((bm2, dout), lambda i: (i, 0)),
        ],
        out_shape=[
            jax.ShapeDtypeStruct((n, dh), jnp.float32),
            jax.ShapeDtypeStruct((n, dout), jnp.float32),
        ],
        compiler_params=pltpu.CompilerParams(
            dimension_semantics=("arbitrary",),
            vmem_limit_bytes=56 * 1024 * 1024,
        ),
    )(adj16, x, y1, y116, W, bg, pt, bp)
    return (h, z)


# speedup vs baseline: 1.1186x; 1.1186x over previous
import functools

import jax
import jax.numpy as jnp
from jax.experimental import pallas as pl
from jax.experimental.pallas import tpu as pltpu


def _body(x_ref, adj_ref, w_ref, bg_ref, pt_ref, bp_ref, h_ref, z_ref, y1_ref,
          *, bm):
    p = pl.program_id(0)
    i = pl.program_id(1)
    a = adj_ref[...]

    @pl.when(p == 0)
    def _pass0():
        y1_ref[pl.ds(i * bm, bm), :] = jnp.dot(
            a, x_ref[...], preferred_element_type=jnp.float32)

    @pl.when(p == 1)
    def _pass1():
        y2 = jnp.dot(a, y1_ref[...], preferred_element_type=jnp.float32)
        xb = x_ref[pl.ds(i * bm, bm), :]
        y1b = y1_ref[pl.ds(i * bm, bm), :]
        xm = (xb + y1b + y2) * (1.0 / 3.0)
        hb = jnp.dot(xm, w_ref[...], preferred_element_type=jnp.float32)
        hb = jnp.maximum(hb + bg_ref[...], 0.0)
        h_ref[...] = hb
        zl = jnp.dot(hb, pt_ref[...], preferred_element_type=jnp.float32)
        zl = zl + bp_ref[...]
        m = jnp.max(zl, axis=1, keepdims=True)
        e = zl - m
        lse = jnp.log(jnp.sum(jnp.exp(e), axis=1, keepdims=True))
        z_ref[...] = e - lse


def kernel(x, adj, W, b_gcn, prototypes, b_pre, t_p):
    n, din = x.shape
    dh = W.shape[1]
    dout = prototypes.shape[0]

    bm = 400
    while n % bm:
        bm //= 2
    nb = n // bm

    inv_t = (1.0 / t_p).astype(jnp.float32) if hasattr(t_p, "astype") else jnp.float32(1.0 / t_p)
    pt = prototypes.T.astype(jnp.float32) * inv_t
    bp = (b_pre.astype(jnp.float32) * inv_t).reshape(1, dout)
    bg = b_gcn.reshape(1, dh)

    h, z = pl.pallas_call(
        functools.partial(_body, bm=bm),
        grid=(2, nb),
        in_specs=[
            pl.BlockSpec((n, din), lambda p, i: (0, 0)),
            pl.BlockSpec((bm, n), lambda p, i: (i, 0)),
            pl.BlockSpec((din, dh), lambda p, i: (0, 0)),
            pl.BlockSpec((1, dh), lambda p, i: (0, 0)),
            pl.BlockSpec((dh, dout), lambda p, i: (0, 0)),
            pl.BlockSpec((1, dout), lambda p, i: (0, 0)),
        ],
        out_specs=[
            pl.BlockSpec((bm, dh), lambda p, i: (p * i, 0)),
            pl.BlockSpec((bm, dout), lambda p, i: (p * i, 0)),
        ],
        out_shape=[
            jax.ShapeDtypeStruct((n, dh), jnp.float32),
            jax.ShapeDtypeStruct((n, dout), jnp.float32),
        ],
        scratch_shapes=[pltpu.VMEM((n, din), jnp.float32)],
        compiler_params=pltpu.CompilerParams(
            dimension_semantics=("arbitrary", "arbitrary"),
            vmem_limit_bytes=100 * 1024 * 1024,
        ),
    )(x, adj, W, bg, pt, bp)
    return (h, z)
